# baseline (device time: 56756 ns/iter reference)
import jax
import jax.numpy as jnp
from jax import lax
from jax.experimental import pallas as pl
from jax.experimental.pallas import tpu as pltpu

N_DEV = 4
BLK = 512


def kernel(x, W1, W2):
    m, _ = x.shape
    _, n = W2.shape

    def body(x_ref, w1_ref, w2_ref, out_ref,
             h_ref, rs_send, rs_recv, ag_recv, acc_ref,
             rs_send_sems, rs_recv_sems, ag_send_sems, ag_recv_sems):
        my = lax.axis_index("i")
        left = lax.rem(my + N_DEV - 1, N_DEV)
        right = lax.rem(my + 1, N_DEV)

        barrier_sem = pltpu.get_barrier_semaphore()
        for nbr in (left, right):
            pl.semaphore_signal(
                barrier_sem, inc=1,
                device_id=(nbr,), device_id_type=pl.DeviceIdType.MESH,
            )
        pl.semaphore_wait(barrier_sem, 2)

        xb = x_ref[...].astype(jnp.bfloat16)
        w1b = w1_ref[...].astype(jnp.bfloat16)
        h_ref[...] = jnp.dot(
            xb, w1b, preferred_element_type=jnp.float32
        ).astype(jnp.bfloat16)

        for s in range(N_DEV - 1):
            c = lax.rem(my + 2 * N_DEV - 1 - s, N_DEV)
            chunk = h_ref[pl.ds(c * BLK, BLK), :]
            if s == 0:
                rs_send[s] = chunk
            else:
                rs_send[s] = chunk + rs_recv[s - 1]
            rdma = pltpu.make_async_remote_copy(
                src_ref=rs_send.at[s],
                dst_ref=rs_recv.at[s],
                send_sem=rs_send_sems.at[s],
                recv_sem=rs_recv_sems.at[s],
                device_id=(right,),
                device_id_type=pl.DeviceIdType.MESH,
            )
            rdma.start()
            rdma.wait()

        acc_ref[...] = (
            h_ref[pl.ds(my * BLK, BLK), :] + rs_recv[N_DEV - 2]
        )

        w2b = w2_ref[...].astype(jnp.bfloat16)
        out_ref[pl.ds(my * BLK, BLK), :] = jnp.dot(
            acc_ref[...], w2b, preferred_element_type=jnp.float32
        )

        for t in range(N_DEV - 1):
            src = acc_ref if t == 0 else ag_recv.at[t - 1]
            rdma = pltpu.make_async_remote_copy(
                src_ref=src,
                dst_ref=ag_recv.at[t],
                send_sem=ag_send_sems.at[t],
                recv_sem=ag_recv_sems.at[t],
                device_id=(right,),
                device_id_type=pl.DeviceIdType.MESH,
            )
            rdma.start()
            rdma.wait()
            origin = lax.rem(my + 2 * N_DEV - 1 - t, N_DEV)
            out_ref[pl.ds(origin * BLK, BLK), :] = jnp.dot(
                ag_recv[t], w2b, preferred_element_type=jnp.float32
            )

    return pl.pallas_call(
        body,
        out_shape=jax.ShapeDtypeStruct((m, n), jnp.float32),
        in_specs=[
            pl.BlockSpec(memory_space=pltpu.VMEM),
            pl.BlockSpec(memory_space=pltpu.VMEM),
            pl.BlockSpec(memory_space=pltpu.VMEM),
        ],
        out_specs=pl.BlockSpec(memory_space=pltpu.VMEM),
        scratch_shapes=[
            pltpu.VMEM((m, BLK), jnp.bfloat16),
            pltpu.VMEM((N_DEV - 1, BLK, BLK), jnp.bfloat16),
            pltpu.VMEM((N_DEV - 1, BLK, BLK), jnp.bfloat16),
            pltpu.VMEM((N_DEV - 1, BLK, BLK), jnp.bfloat16),
            pltpu.VMEM((BLK, BLK), jnp.bfloat16),
            pltpu.SemaphoreType.DMA((N_DEV - 1,)),
            pltpu.SemaphoreType.DMA((N_DEV - 1,)),
            pltpu.SemaphoreType.DMA((N_DEV - 1,)),
            pltpu.SemaphoreType.DMA((N_DEV - 1,)),
        ],
        compiler_params=pltpu.CompilerParams(collective_id=0),
    )(x, W1, W2)


# device time: 38776 ns/iter; 1.4637x vs baseline; 1.4637x over previous
import jax
import jax.numpy as jnp
from jax import lax
from jax.experimental import pallas as pl
from jax.experimental.pallas import tpu as pltpu

N_DEV = 4
BLK = 512
HALF = BLK // 2


def kernel(x, W1, W2):
    m, _ = x.shape
    _, n = W2.shape

    def body(x_ref, w1_ref, w2_ref, out_ref,
             h_ref,
             rsa_send, rsa_recv, rsb_send, rsb_recv,
             aga_recv, agb_recv, acca_ref, accb_ref,
             rsa_ssem, rsa_rsem, rsb_ssem, rsb_rsem,
             aga_ssem, aga_rsem, agb_ssem, agb_rsem):
        my = lax.axis_index("i")
        left = lax.rem(my + N_DEV - 1, N_DEV)
        right = lax.rem(my + 1, N_DEV)

        barrier_sem = pltpu.get_barrier_semaphore()
        for nbr in (left, right):
            pl.semaphore_signal(
                barrier_sem, inc=1,
                device_id=(nbr,), device_id_type=pl.DeviceIdType.MESH,
            )
        pl.semaphore_wait(barrier_sem, 2)

        xb = x_ref[...].astype(jnp.bfloat16)
        w1b = w1_ref[...].astype(jnp.bfloat16)
        h_ref[...] = jnp.dot(
            xb, w1b, preferred_element_type=jnp.float32
        ).astype(jnp.bfloat16)

        def ha(b):
            return h_ref[pl.ds(b * BLK, HALF), :]

        def hb(b):
            return h_ref[pl.ds(b * BLK + HALF, HALF), :]

        def rdma(src, dst, ssem, rsem, dev):
            return pltpu.make_async_remote_copy(
                src_ref=src, dst_ref=dst, send_sem=ssem, recv_sem=rsem,
                device_id=(dev,), device_id_type=pl.DeviceIdType.MESH,
            )

        for s in range(N_DEV - 1):
            ca = lax.rem(my + 2 * N_DEV - 1 - s, N_DEV)
            cb = lax.rem(my + 1 + s, N_DEV)
            if s == 0:
                rsa_send[s] = ha(ca)
                rsb_send[s] = hb(cb)
            else:
                rsa_send[s] = ha(ca) + rsa_recv[s - 1]
                rsb_send[s] = hb(cb) + rsb_recv[s - 1]
            ra = rdma(rsa_send.at[s], rsa_recv.at[s],
                      rsa_ssem.at[s], rsa_rsem.at[s], right)
            rb = rdma(rsb_send.at[s], rsb_recv.at[s],
                      rsb_ssem.at[s], rsb_rsem.at[s], left)
            ra.start()
            rb.start()
            ra.wait()
            rb.wait()

        acca_ref[...] = ha(my) + rsa_recv[N_DEV - 2]
        accb_ref[...] = hb(my) + rsb_recv[N_DEV - 2]

        w2b = w2_ref[...].astype(jnp.bfloat16)

        ags = []
        for t in range(N_DEV - 1):
            srca = acca_ref if t == 0 else aga_recv.at[t - 1]
            srcb = accb_ref if t == 0 else agb_recv.at[t - 1]
            ags.append((
                rdma(srca, aga_recv.at[t], aga_ssem.at[t], aga_rsem.at[t],
                     right),
                rdma(srcb, agb_recv.at[t], agb_ssem.at[t], agb_rsem.at[t],
                     left),
            ))

        def gemm2_own():
            out_ref[pl.ds(my * BLK, HALF), :] = jnp.dot(
                acca_ref[...], w2b, preferred_element_type=jnp.float32
            )
            out_ref[pl.ds(my * BLK + HALF, HALF), :] = jnp.dot(
                accb_ref[...], w2b, preferred_element_type=jnp.float32
            )

        def gemm2_hop(t):
            origin_a = lax.rem(my + 2 * N_DEV - 1 - t, N_DEV)
            origin_b = lax.rem(my + 1 + t, N_DEV)
            out_ref[pl.ds(origin_a * BLK, HALF), :] = jnp.dot(
                aga_recv[t], w2b, preferred_element_type=jnp.float32
            )
            out_ref[pl.ds(origin_b * BLK + HALF, HALF), :] = jnp.dot(
                agb_recv[t], w2b, preferred_element_type=jnp.float32
            )

        ags[0][0].start()
        ags[0][1].start()
        gemm2_own()
        for t in range(N_DEV - 1):
            ags[t][0].wait_recv()
            ags[t][1].wait_recv()
            if t + 1 < N_DEV - 1:
                ags[t + 1][0].start()
                ags[t + 1][1].start()
            gemm2_hop(t)
        for t in range(N_DEV - 1):
            ags[t][0].wait_send()
            ags[t][1].wait_send()

    return pl.pallas_call(
        body,
        out_shape=jax.ShapeDtypeStruct((m, n), jnp.float32),
        in_specs=[
            pl.BlockSpec(memory_space=pltpu.VMEM),
            pl.BlockSpec(memory_space=pltpu.VMEM),
            pl.BlockSpec(memory_space=pltpu.VMEM),
        ],
        out_specs=pl.BlockSpec(memory_space=pltpu.VMEM),
        scratch_shapes=[
            pltpu.VMEM((m, BLK), jnp.bfloat16),
            pltpu.VMEM((N_DEV - 1, HALF, BLK), jnp.bfloat16),
            pltpu.VMEM((N_DEV - 1, HALF, BLK), jnp.bfloat16),
            pltpu.VMEM((N_DEV - 1, HALF, BLK), jnp.bfloat16),
            pltpu.VMEM((N_DEV - 1, HALF, BLK), jnp.bfloat16),
            pltpu.VMEM((N_DEV - 1, HALF, BLK), jnp.bfloat16),
            pltpu.VMEM((N_DEV - 1, HALF, BLK), jnp.bfloat16),
            pltpu.VMEM((HALF, BLK), jnp.bfloat16),
            pltpu.VMEM((HALF, BLK), jnp.bfloat16),
            pltpu.SemaphoreType.DMA((N_DEV - 1,)),
            pltpu.SemaphoreType.DMA((N_DEV - 1,)),
            pltpu.SemaphoreType.DMA((N_DEV - 1,)),
            pltpu.SemaphoreType.DMA((N_DEV - 1,)),
            pltpu.SemaphoreType.DMA((N_DEV - 1,)),
            pltpu.SemaphoreType.DMA((N_DEV - 1,)),
            pltpu.SemaphoreType.DMA((N_DEV - 1,)),
            pltpu.SemaphoreType.DMA((N_DEV - 1,)),
        ],
        compiler_params=pltpu.CompilerParams(collective_id=0),
    )(x, W1, W2)


# device time: 30130 ns/iter; 1.8837x vs baseline; 1.2870x over previous
import jax
import jax.numpy as jnp
from jax import lax
from jax.experimental import pallas as pl
from jax.experimental.pallas import tpu as pltpu

N_DEV = 4
BLK = 512
HALF = BLK // 2
NSUB = 4
SUB = HALF // NSUB
NHOP = N_DEV - 1


def kernel(x, W1, W2):
    m, _ = x.shape
    _, n = W2.shape

    def body(x_ref, w1_ref, w2_ref, out_ref,
             h_ref,
             rsa_send, rsa_recv, rsb_send, rsb_recv,
             aga_recv, agb_recv, acca_ref, accb_ref,
             rsa_ssem, rsa_rsem, rsb_ssem, rsb_rsem,
             aga_ssem, aga_rsem, agb_ssem, agb_rsem):
        my = lax.axis_index("i")
        left = lax.rem(my + N_DEV - 1, N_DEV)
        right = lax.rem(my + 1, N_DEV)

        barrier_sem = pltpu.get_barrier_semaphore()
        for nbr in (left, right):
            pl.semaphore_signal(
                barrier_sem, inc=1,
                device_id=(nbr,), device_id_type=pl.DeviceIdType.MESH,
            )
        pl.semaphore_wait(barrier_sem, 2)

        w1b = w1_ref[...].astype(jnp.bfloat16)

        def rdma(src, dst, ssem, rsem, dev):
            return pltpu.make_async_remote_copy(
                src_ref=src, dst_ref=dst, send_sem=ssem, recv_sem=rsem,
                device_id=(dev,), device_id_type=pl.DeviceIdType.MESH,
            )

        def arow(b, j):
            return b * BLK + j * SUB

        def brow(b, j):
            return b * BLK + HALF + j * SUB

        def h_sub(row):
            return h_ref[pl.ds(row, SUB), :]

        def ca(s):
            return lax.rem(my + 2 * N_DEV - 1 - s, N_DEV)

        def cb(s):
            return lax.rem(my + 1 + s, N_DEV)

        xb16 = None

        def gemm1_rows(row):
            xsub = x_ref[pl.ds(row, SUB), :].astype(jnp.bfloat16)
            return jnp.dot(
                xsub, w1b, preferred_element_type=jnp.float32
            ).astype(jnp.bfloat16)

        rs = [[None] * NSUB for _ in range(NHOP)]
        rsb_ = [[None] * NSUB for _ in range(NHOP)]
        for j in range(NSUB):
            rsa_send[0, j] = gemm1_rows(arow(ca(0), j))
            rs[0][j] = rdma(rsa_send.at[0, j], rsa_recv.at[0, j],
                            rsa_ssem.at[0, j], rsa_rsem.at[0, j], right)
            rs[0][j].start()
            rsb_send[0, j] = gemm1_rows(brow(cb(0), j))
            rsb_[0][j] = rdma(rsb_send.at[0, j], rsb_recv.at[0, j],
                              rsb_ssem.at[0, j], rsb_rsem.at[0, j], left)
            rsb_[0][j].start()

        xb16 = x_ref[...].astype(jnp.bfloat16)
        h_ref[...] = jnp.dot(
            xb16, w1b, preferred_element_type=jnp.float32
        ).astype(jnp.bfloat16)

        for s in range(1, NHOP):
            for j in range(NSUB):
                rs[s - 1][j].wait_recv()
                rsa_send[s, j] = h_sub(arow(ca(s), j)) + rsa_recv[s - 1, j]
                rs[s][j] = rdma(rsa_send.at[s, j], rsa_recv.at[s, j],
                                rsa_ssem.at[s, j], rsa_rsem.at[s, j], right)
                rs[s][j].start()
                rsb_[s - 1][j].wait_recv()
                rsb_send[s, j] = h_sub(brow(cb(s), j)) + rsb_recv[s - 1, j]
                rsb_[s][j] = rdma(rsb_send.at[s, j], rsb_recv.at[s, j],
                                  rsb_ssem.at[s, j], rsb_rsem.at[s, j], left)
                rsb_[s][j].start()

        aga = [[None] * NSUB for _ in range(NHOP)]
        agb = [[None] * NSUB for _ in range(NHOP)]
        for j in range(NSUB):
            rs[NHOP - 1][j].wait_recv()
            acca_ref[j] = h_sub(arow(my, j)) + rsa_recv[NHOP - 1, j]
            aga[0][j] = rdma(acca_ref.at[j], aga_recv.at[0, j],
                             aga_ssem.at[0, j], aga_rsem.at[0, j], right)
            aga[0][j].start()
            rsb_[NHOP - 1][j].wait_recv()
            accb_ref[j] = h_sub(brow(my, j)) + rsb_recv[NHOP - 1, j]
            agb[0][j] = rdma(accb_ref.at[j], agb_recv.at[0, j],
                             agb_ssem.at[0, j], agb_rsem.at[0, j], left)
            agb[0][j].start()

        w2b = w2_ref[...].astype(jnp.bfloat16)
        out_ref[pl.ds(my * BLK, HALF), :] = jnp.dot(
            acca_ref[...].reshape(HALF, BLK), w2b,
            preferred_element_type=jnp.float32,
        )
        out_ref[pl.ds(my * BLK + HALF, HALF), :] = jnp.dot(
            accb_ref[...].reshape(HALF, BLK), w2b,
            preferred_element_type=jnp.float32,
        )

        for t in range(NHOP):
            for j in range(NSUB):
                aga[t][j].wait_recv()
                agb[t][j].wait_recv()
                if t + 1 < NHOP:
                    aga[t + 1][j] = rdma(
                        aga_recv.at[t, j], aga_recv.at[t + 1, j],
                        aga_ssem.at[t + 1, j], aga_rsem.at[t + 1, j], right)
                    aga[t + 1][j].start()
                    agb[t + 1][j] = rdma(
                        agb_recv.at[t, j], agb_recv.at[t + 1, j],
                        agb_ssem.at[t + 1, j], agb_rsem.at[t + 1, j], left)
                    agb[t + 1][j].start()
            out_ref[pl.ds(ca(t) * BLK, HALF), :] = jnp.dot(
                aga_recv[t].reshape(HALF, BLK), w2b,
                preferred_element_type=jnp.float32,
            )
            out_ref[pl.ds(cb(t) * BLK + HALF, HALF), :] = jnp.dot(
                agb_recv[t].reshape(HALF, BLK), w2b,
                preferred_element_type=jnp.float32,
            )

        for s in range(NHOP):
            for j in range(NSUB):
                rs[s][j].wait_send()
                rsb_[s][j].wait_send()
                aga[s][j].wait_send()
                agb[s][j].wait_send()

    return pl.pallas_call(
        body,
        out_shape=jax.ShapeDtypeStruct((m, n), jnp.float32),
        in_specs=[
            pl.BlockSpec(memory_space=pltpu.VMEM),
            pl.BlockSpec(memory_space=pltpu.VMEM),
            pl.BlockSpec(memory_space=pltpu.VMEM),
        ],
        out_specs=pl.BlockSpec(memory_space=pltpu.VMEM),
        scratch_shapes=[
            pltpu.VMEM((m, BLK), jnp.bfloat16),
            pltpu.VMEM((NHOP, NSUB, SUB, BLK), jnp.bfloat16),
            pltpu.VMEM((NHOP, NSUB, SUB, BLK), jnp.bfloat16),
            pltpu.VMEM((NHOP, NSUB, SUB, BLK), jnp.bfloat16),
            pltpu.VMEM((NHOP, NSUB, SUB, BLK), jnp.bfloat16),
            pltpu.VMEM((NHOP, NSUB, SUB, BLK), jnp.bfloat16),
            pltpu.VMEM((NHOP, NSUB, SUB, BLK), jnp.bfloat16),
            pltpu.VMEM((NSUB, SUB, BLK), jnp.bfloat16),
            pltpu.VMEM((NSUB, SUB, BLK), jnp.bfloat16),
            pltpu.SemaphoreType.DMA((NHOP, NSUB)),
            pltpu.SemaphoreType.DMA((NHOP, NSUB)),
            pltpu.SemaphoreType.DMA((NHOP, NSUB)),
            pltpu.SemaphoreType.DMA((NHOP, NSUB)),
            pltpu.SemaphoreType.DMA((NHOP, NSUB)),
            pltpu.SemaphoreType.DMA((NHOP, NSUB)),
            pltpu.SemaphoreType.DMA((NHOP, NSUB)),
            pltpu.SemaphoreType.DMA((NHOP, NSUB)),
        ],
        compiler_params=pltpu.CompilerParams(collective_id=0),
    )(x, W1, W2)
